# baseline (speedup 1.0000x reference)
import functools

import jax
import jax.numpy as jnp
from jax import lax
from jax.experimental import pallas as pl
from jax.experimental.pallas import tpu as pltpu
from jax.experimental.pallas import tpu_sc as plsc

LANES = 16


def _sc_embed(x_flat, token_table, pos2, *, L, D, NC, NS, BLK):
    N = x_flat.shape[0]
    NW = NC * NS
    n_per_w = N // NW
    n_blk = n_per_w // BLK

    mesh = plsc.VectorSubcoreMesh(core_axis_name="c", subcore_axis_name="s")

    @functools.partial(
        pl.kernel,
        out_type=jax.ShapeDtypeStruct((N, D), jnp.float32),
        mesh=mesh,
        scratch_types=[
            pltpu.VMEM((2 * L, D), jnp.float32),
            pltpu.VMEM((BLK,), jnp.int32),
            pltpu.VMEM((BLK, D), jnp.float32),
            pltpu.SemaphoreType.DMA,
        ],
        compiler_params=pltpu.CompilerParams(use_tc_tiling_on_sc=False),
    )
    def body(x_hbm, tok_hbm, pos2_hbm, out_hbm, pos_vm, idx_vm, rows_vm, sem):
        wid = lax.axis_index("c") * NS + lax.axis_index("s")
        pltpu.sync_copy(pos2_hbm, pos_vm)
        wbase = wid * n_per_w

        def blk_body(b, carry):
            base = wbase + b * BLK
            pltpu.sync_copy(x_hbm.at[pl.ds(base, BLK)], idx_vm)
            pltpu.async_copy(tok_hbm.at[idx_vm], rows_vm, sem).wait()
            start = lax.rem(base, L)

            def row_body(r, c):
                pr = start + r
                rows_vm[r, pl.ds(0, LANES)] = (
                    rows_vm[r, pl.ds(0, LANES)] + pos_vm[pr, pl.ds(0, LANES)]
                )
                rows_vm[r, pl.ds(LANES, LANES)] = (
                    rows_vm[r, pl.ds(LANES, LANES)]
                    + pos_vm[pr, pl.ds(LANES, LANES)]
                )
                return c

            lax.fori_loop(0, BLK, row_body, 0)
            pltpu.sync_copy(rows_vm, out_hbm.at[pl.ds(base, BLK)])
            return carry

        lax.fori_loop(0, n_blk, blk_body, 0)

    return body(x_flat, token_table, pos2)


def kernel(x, token_table, pos_table):
    B, L = x.shape
    V, D = token_table.shape
    N = B * L

    info = plsc.get_sparse_core_info()
    NC, NS = info.num_cores, info.num_subcores

    x_flat = x.reshape(N).astype(jnp.int32)
    pos2 = jnp.concatenate([pos_table, pos_table], axis=0)

    out = _sc_embed(x_flat, token_table, pos2, L=L, D=D, NC=NC, NS=NS, BLK=128)
    return out.reshape(B, L, D)


# speedup vs baseline: 1.1020x; 1.1020x over previous
import functools

import jax
import jax.numpy as jnp
from jax import lax
from jax.experimental import pallas as pl
from jax.experimental.pallas import tpu as pltpu
from jax.experimental.pallas import tpu_sc as plsc


def _sc_embed(x2d, token_table, pos2, *, L, D, NC, NS, BLK, KBUF):
    n_rows = x2d.shape[0]
    NW = NC * NS
    rows_per_w = n_rows // NW
    n_grp = rows_per_w // KBUF
    N = n_rows * BLK

    mesh = plsc.VectorSubcoreMesh(core_axis_name="c", subcore_axis_name="s")

    @functools.partial(
        pl.kernel,
        out_type=jax.ShapeDtypeStruct((N, D), jnp.float32),
        mesh=mesh,
        scratch_types=[
            pltpu.VMEM((rows_per_w, BLK), jnp.int32),
            *[pltpu.VMEM((BLK, D), jnp.float32) for _ in range(KBUF)],
            pltpu.SemaphoreType.DMA,
            pltpu.SemaphoreType.DMA,
            pltpu.SemaphoreType.DMA,
        ],
        compiler_params=pltpu.CompilerParams(use_tc_tiling_on_sc=False),
    )
    def body(x_hbm, tok_hbm, pos2_hbm, out_hbm, idx_vm, *rest):
        bufs = rest[:KBUF]
        sem_f, sem_g, sem_o = rest[KBUF:]
        wid = lax.axis_index("c") * NS + lax.axis_index("s")
        pltpu.sync_copy(x_hbm.at[pl.ds(wid * rows_per_w, rows_per_w)], idx_vm)
        row0 = wid * rows_per_w

        def grp(g, carry):
            fills = []
            for k in range(KBUF):
                b = g * KBUF + k
                start = lax.rem((row0 + b) * BLK, L)
                fills.append(
                    pltpu.async_copy(pos2_hbm.at[pl.ds(start, BLK)], bufs[k], sem_f)
                )
            gathers = []
            for k in range(KBUF):
                b = g * KBUF + k
                fills[k].wait()
                gathers.append(
                    pltpu.async_copy(tok_hbm.at[idx_vm.at[b]], bufs[k], sem_g, add=True)
                )
            for gd in gathers:
                gd.wait()
            outs = []
            for k in range(KBUF):
                b = g * KBUF + k
                outs.append(
                    pltpu.async_copy(
                        bufs[k], out_hbm.at[pl.ds((row0 + b) * BLK, BLK)], sem_o
                    )
                )
            for od in outs:
                od.wait()
            return carry

        lax.fori_loop(0, n_grp, grp, 0)

    return body(x2d, token_table, pos2)


def kernel(x, token_table, pos_table):
    B, L = x.shape
    V, D = token_table.shape
    N = B * L

    info = plsc.get_sparse_core_info()
    NC, NS = info.num_cores, info.num_subcores

    BLK = 128
    x2d = x.reshape(N // BLK, BLK).astype(jnp.int32)
    pos2 = jnp.concatenate([pos_table, pos_table], axis=0)

    out = _sc_embed(x2d, token_table, pos2, L=L, D=D, NC=NC, NS=NS, BLK=BLK, KBUF=10)
    return out.reshape(B, L, D)


# speedup vs baseline: 1.1924x; 1.0820x over previous
import functools

import jax
import jax.numpy as jnp
from jax import lax
from jax.experimental import pallas as pl
from jax.experimental.pallas import tpu as pltpu
from jax.experimental.pallas import tpu_sc as plsc


def _sc_embed(x2d, token_table, pos_table, pidx, *, L, D, NC, NS, BLK, KBUF, GW):
    n_rows = x2d.shape[0]
    NW = NC * NS
    rows_per_w = n_rows // NW
    n_pidx = pidx.shape[0]
    N = n_rows * BLK

    mesh = plsc.VectorSubcoreMesh(core_axis_name="c", subcore_axis_name="s")

    @functools.partial(
        pl.kernel,
        out_type=jax.ShapeDtypeStruct((N, D), jnp.float32),
        mesh=mesh,
        scratch_types=[
            pltpu.VMEM((rows_per_w, BLK), jnp.int32),
            pltpu.VMEM((n_pidx, BLK), jnp.int32),
            pltpu.VMEM_SHARED((L, D), jnp.float32),
            *[pltpu.VMEM((BLK, D), jnp.float32) for _ in range(KBUF)],
            pltpu.SemaphoreType.DMA((KBUF,)),
            pltpu.SemaphoreType.DMA((KBUF,)),
            pltpu.SemaphoreType.DMA((KBUF,)),
        ],
        compiler_params=pltpu.CompilerParams(use_tc_tiling_on_sc=False),
    )
    def body(x_hbm, tok_hbm, pos_hbm, pidx_hbm, out_hbm, idx_vm, pidx_vm,
             pos_sh, *rest):
        bufs = rest[:KBUF]
        sem_g, sem_a, sem_o = rest[KBUF:]
        wid = lax.axis_index("c") * NS + lax.axis_index("s")
        pltpu.sync_copy(x_hbm.at[pl.ds(wid * rows_per_w, rows_per_w)], idx_vm)
        pltpu.sync_copy(pidx_hbm, pidx_vm)
        pltpu.sync_copy(pos_hbm, pos_sh)
        row0 = wid * rows_per_w

        gh = [None] * KBUF
        ah = [None] * KBUF
        oh = [None] * KBUF
        AW = 1

        for t in range(rows_per_w + GW + AW):
            if t < rows_per_w:
                k = t % KBUF
                if oh[k] is not None:
                    oh[k].wait()
                gh[k] = pltpu.async_copy(
                    tok_hbm.at[idx_vm.at[t]], bufs[k], sem_g.at[k]
                )
            j = t - GW
            if 0 <= j < rows_per_w:
                kj = j % KBUF
                gh[kj].wait()
                ah[kj] = pltpu.async_copy(
                    pos_sh.at[pidx_vm.at[j % n_pidx]], bufs[kj],
                    sem_a.at[kj], add=True
                )
            i = t - GW - AW
            if 0 <= i < rows_per_w:
                ki = i % KBUF
                ah[ki].wait()
                oh[ki] = pltpu.async_copy(
                    bufs[ki], out_hbm.at[pl.ds((row0 + i) * BLK, BLK)],
                    sem_o.at[ki]
                )
        for k in range(KBUF):
            if oh[k] is not None:
                oh[k].wait()

    return body(x2d, token_table, pos_table, pidx)


def kernel(x, token_table, pos_table):
    B, L = x.shape
    V, D = token_table.shape
    N = B * L

    info = plsc.get_sparse_core_info()
    NC, NS = info.num_cores, info.num_subcores

    BLK = 128
    x2d = x.reshape(N // BLK, BLK).astype(jnp.int32)
    import math

    period = L // math.gcd(BLK, L)
    pidx = (
        (jnp.arange(period * BLK, dtype=jnp.int32) % L).reshape(period, BLK)
    )

    out = _sc_embed(
        x2d, token_table, pos_table, pidx,
        L=L, D=D, NC=NC, NS=NS, BLK=BLK, KBUF=10, GW=7,
    )
    return out.reshape(B, L, D)


# speedup vs baseline: 1.1935x; 1.0009x over previous
import functools

import jax
import jax.numpy as jnp
from jax import lax
from jax.experimental import pallas as pl
from jax.experimental.pallas import tpu as pltpu
from jax.experimental.pallas import tpu_sc as plsc


def _sc_embed(x, token_table, pos_table, pidx, *, B, L, D, NC, NS, KBUF, GW):
    NW = NC * NS
    seq_per_w = B // NW

    mesh = plsc.VectorSubcoreMesh(core_axis_name="c", subcore_axis_name="s")

    @functools.partial(
        pl.kernel,
        out_type=jax.ShapeDtypeStruct((B, L, D), jnp.float32),
        mesh=mesh,
        scratch_types=[
            pltpu.VMEM((seq_per_w, L), jnp.int32),
            pltpu.VMEM((L,), jnp.int32),
            pltpu.VMEM_SHARED((L, D), jnp.float32),
            *[pltpu.VMEM((L, D), jnp.float32) for _ in range(KBUF)],
            pltpu.SemaphoreType.DMA((KBUF,)),
            pltpu.SemaphoreType.DMA((KBUF,)),
            pltpu.SemaphoreType.DMA((KBUF,)),
        ],
        compiler_params=pltpu.CompilerParams(use_tc_tiling_on_sc=False),
    )
    def body(x_hbm, tok_hbm, pos_hbm, pidx_hbm, out_hbm, idx_vm, pidx_vm,
             pos_sh, *rest):
        bufs = rest[:KBUF]
        sem_g, sem_a, sem_o = rest[KBUF:]
        wid = lax.axis_index("c") * NS + lax.axis_index("s")
        pltpu.sync_copy(x_hbm.at[pl.ds(wid * seq_per_w, seq_per_w)], idx_vm)
        pltpu.sync_copy(pidx_hbm, pidx_vm)
        pltpu.sync_copy(pos_hbm, pos_sh)
        seq0 = wid * seq_per_w

        gh = [None] * KBUF
        ah = [None] * KBUF
        oh = [None] * KBUF
        AW = 1

        for t in range(seq_per_w + GW + AW):
            if t < seq_per_w:
                k = t % KBUF
                if oh[k] is not None:
                    oh[k].wait()
                gh[k] = pltpu.async_copy(
                    tok_hbm.at[idx_vm.at[t]], bufs[k], sem_g.at[k]
                )
            j = t - GW
            if 0 <= j < seq_per_w:
                kj = j % KBUF
                gh[kj].wait()
                ah[kj] = pltpu.async_copy(
                    pos_sh.at[pidx_vm], bufs[kj], sem_a.at[kj], add=True
                )
            i = t - GW - AW
            if 0 <= i < seq_per_w:
                ki = i % KBUF
                ah[ki].wait()
                oh[ki] = pltpu.async_copy(
                    bufs[ki], out_hbm.at[seq0 + i], sem_o.at[ki]
                )
        for k in range(KBUF):
            if oh[k] is not None:
                oh[k].wait()

    return body(x, token_table, pos_table, pidx)


def kernel(x, token_table, pos_table):
    B, L = x.shape
    V, D = token_table.shape

    info = plsc.get_sparse_core_info()
    NC, NS = info.num_cores, info.num_subcores

    pidx = jnp.arange(L, dtype=jnp.int32)

    out = _sc_embed(
        x.astype(jnp.int32), token_table, pos_table, pidx,
        B=B, L=L, D=D, NC=NC, NS=NS, KBUF=8, GW=6,
    )
    return out
